# baseline (speedup 1.0000x reference)
import jax
import jax.numpy as jnp
from jax.experimental import pallas as pl
from jax.experimental.pallas import tpu as pltpu

DIM = 128
BALL = 128
N = 2048
NB = N // BALL
HID = DIM * 4
EPS = float(jnp.finfo(jnp.float32).eps)
SCALE = DIM ** -0.5
NEG = -1e30

_DN = (((1,), (1,)), ((), ()))


def _prep_body(x_ref, pos_ref, n1_ref, x1_ref, sel_ref):
    x = x_ref[...]
    p = pos_ref[...].reshape(NB, BALL)
    rel = (p - jnp.mean(p, axis=1, keepdims=True)).reshape(N, 1)
    v = jnp.mean(x * x, axis=-1, keepdims=True)
    x1 = x * jax.lax.rsqrt(v + EPS) * n1_ref[...] + rel
    x1_ref[...] = x1
    bm = jnp.mean(x1.reshape(NB, BALL, DIM), axis=1)
    simT = jax.lax.dot_general(bm, x1, _DN,
                               preferred_element_type=jnp.float32)
    idx = jax.lax.broadcasted_iota(jnp.int32, (NB, N), 0)
    m1 = jnp.max(simT, axis=0, keepdims=True)
    i1 = jnp.min(jnp.where(simT == m1, idx, NB), axis=0, keepdims=True)
    sel1 = idx == i1
    sim2 = jnp.where(sel1, NEG, simT)
    m2 = jnp.max(sim2, axis=0, keepdims=True)
    i2 = jnp.min(jnp.where(sim2 == m2, idx, NB), axis=0, keepdims=True)
    sel_ref[...] = jnp.where(sel1 | (idx == i2), 1.0,
                             0.0).astype(jnp.bfloat16)


def _attn_body(x_ref, x1in_ref, selin_ref, n2_ref, w1w_ref, w1b_ref,
               w2w_ref, w2b_ref, w3w_ref, w3b_ref, o_ref):
    x1 = x1in_ref[...]
    idx = jax.lax.broadcasted_iota(jnp.int32, (NB, N), 0)
    cc = jax.lax.broadcasted_iota(jnp.int32, (NB, N), 1) // BALL
    cmap = (idx == cc).astype(jnp.bfloat16)
    mask = jax.lax.dot_general(selin_ref[...], cmap,
                               (((0,), (0,)), ((), ())),
                               preferred_element_type=jnp.float32)
    s = jax.lax.dot_general(x1 * SCALE, x1, _DN,
                            preferred_element_type=jnp.float32)
    p = jnp.exp(s) * mask
    attn = jax.lax.dot_general(p, x1, (((1,), (0,)), ((), ())),
                               preferred_element_type=jnp.float32)
    attn = attn / jnp.sum(p, axis=-1, keepdims=True)
    x2 = x_ref[...] + attn
    v2 = jnp.mean(x2 * x2, axis=-1, keepdims=True)
    xn = x2 * jax.lax.rsqrt(v2 + EPS) * n2_ref[...]
    a = jax.lax.dot_general(xn, w1w_ref[...], _DN,
                            preferred_element_type=jnp.float32) + w1b_ref[...]
    b = jax.lax.dot_general(xn, w2w_ref[...], _DN,
                            preferred_element_type=jnp.float32) + w2b_ref[...]
    h = b * (a * jax.nn.sigmoid(a))
    o_ref[...] = x2 + jax.lax.dot_general(
        h, w3w_ref[...], _DN, preferred_element_type=jnp.float32) + w3b_ref[...]


def kernel(x, pos, batch_idx, norm1_w, norm2_w, w1_w, w1_b, w2_w, w2_b,
           w3_w, w3_b):
    del batch_idx
    x1, sel = pl.pallas_call(
        _prep_body,
        out_shape=(jax.ShapeDtypeStruct((N, DIM), jnp.float32),
                   jax.ShapeDtypeStruct((NB, N), jnp.bfloat16)),
    )(x, pos, norm1_w.reshape(1, DIM))
    out = pl.pallas_call(
        _attn_body,
        out_shape=jax.ShapeDtypeStruct((N, DIM), jnp.float32),
    )(x, x1, sel, norm2_w.reshape(1, DIM),
      w1_w, w1_b.reshape(1, HID), w2_w, w2_b.reshape(1, HID),
      w3_w, w3_b.reshape(1, DIM))
    return out
